# baseline (speedup 1.0000x reference)
import functools

import jax
import jax.numpy as jnp
from jax import lax
from jax.experimental import pallas as pl
from jax.experimental.pallas import tpu as pltpu
from jax.experimental.pallas import tpu_sc as plsc

ROWS = 8192
DIM = 4096
LANES = 16
NUM_CORES = 2
NUM_SUBCORES = 16
NW = NUM_CORES * NUM_SUBCORES
ROWS_PER_W = ROWS // NW
BLK = 8
NBLK = ROWS_PER_W // BLK
NS = NBLK // 2
NCHUNK = DIM // LANES
NQ = 4
QTR = DIM // NQ
NQCHUNK = NCHUNK // NQ
U = 8


def _permute_body(x_hbm, idx_hbm, out_hbm, idx_v,
                  in_a, in_b, out_h0, out_h1, si_a, si_b, so_h0, so_h1):
    wid = lax.axis_index("s") * NUM_CORES + lax.axis_index("c")
    base = wid * ROWS_PER_W
    pltpu.sync_copy(idx_hbm, idx_v)

    def in_slice(b):
        return x_hbm.at[pl.ds(base + b * BLK, BLK)]

    def out_slice(b, q):
        return out_hbm.at[pl.ds(base + b * BLK, BLK), pl.ds(q * QTR, QTR)]

    def gather_quarter(in_ref, out_ref, q):
        @plsc.parallel_loop(0, NQCHUNK, step=1, unroll=U)
        def _(j):
            cv = idx_v[pl.ds((q * NQCHUNK + j) * LANES, LANES)]
            for r in range(BLK):
                rv = jnp.full((LANES,), r, jnp.int32)
                vals = plsc.load_gather(in_ref, [rv, cv])
                out_ref[r, pl.ds(j * LANES, LANES)] = vals

    def wait_in(buf, sem):
        pltpu.make_async_copy(in_slice(0), buf, sem).wait()

    def wait_out(buf, sem):
        pltpu.make_async_copy(buf, out_slice(0, 0), sem).wait()

    out_bufs = (out_h0, out_h1)
    out_sems = (so_h0, so_h1)

    def do_block(b, in_buf, in_sem, first):
        for q in range(NQ):
            buf, sem = out_bufs[q % 2], out_sems[q % 2]
            if not (first and q < 2):
                wait_out(buf, sem)
            gather_quarter(in_buf, buf, q)
            pltpu.async_copy(buf, out_slice(b, q), sem)

    pltpu.async_copy(in_slice(0), in_a, si_a)
    pltpu.async_copy(in_slice(1), in_b, si_b)

    wait_in(in_a, si_a)
    do_block(0, in_a, si_a, first=True)
    pltpu.async_copy(in_slice(2), in_a, si_a)
    wait_in(in_b, si_b)
    do_block(1, in_b, si_b, first=False)

    def super_body(s, c):
        b0 = 2 * s

        @pl.when(b0 + 1 < NBLK)
        def _():
            pltpu.async_copy(in_slice(b0 + 1), in_b, si_b)

        wait_in(in_a, si_a)
        do_block(b0, in_a, si_a, first=False)

        @pl.when(b0 + 2 < NBLK)
        def _():
            pltpu.async_copy(in_slice(b0 + 2), in_a, si_a)

        wait_in(in_b, si_b)
        do_block(b0 + 1, in_b, si_b, first=False)
        return c

    lax.fori_loop(1, NS, super_body, 0)

    wait_out(out_h0, so_h0)
    wait_out(out_h1, so_h1)


@jax.jit
def _permute(x, perm):
    mesh = plsc.VectorSubcoreMesh(core_axis_name="c", subcore_axis_name="s")
    f = functools.partial(
        pl.kernel,
        mesh=mesh,
        out_type=jax.ShapeDtypeStruct((ROWS, DIM), jnp.float32),
        scratch_types=[
            pltpu.VMEM((DIM,), jnp.int32),
            pltpu.VMEM((BLK, DIM), jnp.float32),
            pltpu.VMEM((BLK, DIM), jnp.float32),
            pltpu.VMEM((BLK, QTR), jnp.float32),
            pltpu.VMEM((BLK, QTR), jnp.float32),
            pltpu.SemaphoreType.DMA,
            pltpu.SemaphoreType.DMA,
            pltpu.SemaphoreType.DMA,
            pltpu.SemaphoreType.DMA,
        ],
        compiler_params=pltpu.CompilerParams(needs_layout_passes=False),
    )(_permute_body)
    return f(x, perm)


def kernel(x, shuffled_indices, inverse_indices, reverse):
    perm = jnp.where(jnp.asarray(reverse), inverse_indices, shuffled_indices)
    y = _permute(x, perm)
    objective = jnp.zeros((), dtype=jnp.float32)
    return (y, objective)


# speedup vs baseline: 1.0227x; 1.0227x over previous
import functools

import jax
import jax.numpy as jnp
from jax import lax
from jax.experimental import pallas as pl
from jax.experimental.pallas import tpu as pltpu
from jax.experimental.pallas import tpu_sc as plsc

ROWS = 8192
DIM = 4096
LANES = 16
NUM_CORES = 2
NUM_SUBCORES = 16
NW = NUM_CORES * NUM_SUBCORES
ROWS_PER_W = ROWS // NW
BLK = 8
NBLK = ROWS_PER_W // BLK
NS = NBLK // 2
NCHUNK = DIM // LANES
HALF = DIM // 2
NHCHUNK = NCHUNK // 2
U = 8


def _permute_body(x_hbm, idx_hbm, out_hbm, idx_v,
                  in_a, in_b, out_h0, out_h1, si_a, si_b, so_h0, so_h1):
    wid = lax.axis_index("s") * NUM_CORES + lax.axis_index("c")
    pltpu.sync_copy(idx_hbm, idx_v)

    def in_slice(b):
        return x_hbm.at[pl.ds((b * NW + wid) * BLK, BLK)]

    def out_slice(b, half):
        return out_hbm.at[pl.ds((b * NW + wid) * BLK, BLK),
                          pl.ds(half * HALF, HALF)]

    def gather_half(in_ref, out_ref, half):
        @plsc.parallel_loop(0, NHCHUNK, step=1, unroll=U)
        def _(j):
            cv = idx_v[pl.ds((half * NHCHUNK + j) * LANES, LANES)]
            for r in range(BLK):
                rv = jnp.full((LANES,), r, jnp.int32)
                vals = plsc.load_gather(in_ref, [rv, cv])
                out_ref[r, pl.ds(j * LANES, LANES)] = vals

    def wait_in(buf, sem):
        pltpu.make_async_copy(in_slice(0), buf, sem).wait()

    def wait_out(buf, half, sem):
        pltpu.make_async_copy(buf, out_slice(0, half), sem).wait()

    def do_block(b, in_buf, in_sem, first):
        if not first:
            wait_out(out_h0, 0, so_h0)
        gather_half(in_buf, out_h0, 0)
        pltpu.async_copy(out_h0, out_slice(b, 0), so_h0)
        if not first:
            wait_out(out_h1, 1, so_h1)
        gather_half(in_buf, out_h1, 1)
        pltpu.async_copy(out_h1, out_slice(b, 1), so_h1)

    pltpu.async_copy(in_slice(0), in_a, si_a)
    pltpu.async_copy(in_slice(1), in_b, si_b)

    wait_in(in_a, si_a)
    do_block(0, in_a, si_a, first=True)
    pltpu.async_copy(in_slice(2), in_a, si_a)
    wait_in(in_b, si_b)
    do_block(1, in_b, si_b, first=False)

    def super_body(s, c):
        b0 = 2 * s

        @pl.when(b0 + 1 < NBLK)
        def _():
            pltpu.async_copy(in_slice(b0 + 1), in_b, si_b)

        wait_in(in_a, si_a)
        do_block(b0, in_a, si_a, first=False)

        @pl.when(b0 + 2 < NBLK)
        def _():
            pltpu.async_copy(in_slice(b0 + 2), in_a, si_a)

        wait_in(in_b, si_b)
        do_block(b0 + 1, in_b, si_b, first=False)
        return c

    lax.fori_loop(1, NS, super_body, 0)

    pltpu.make_async_copy(out_h0, out_slice(0, 0), so_h0).wait()
    pltpu.make_async_copy(out_h1, out_slice(0, 1), so_h1).wait()


@jax.jit
def _permute(x, perm):
    mesh = plsc.VectorSubcoreMesh(core_axis_name="c", subcore_axis_name="s")
    f = functools.partial(
        pl.kernel,
        mesh=mesh,
        out_type=jax.ShapeDtypeStruct((ROWS, DIM), jnp.float32),
        scratch_types=[
            pltpu.VMEM((DIM,), jnp.int32),
            pltpu.VMEM((BLK, DIM), jnp.float32),
            pltpu.VMEM((BLK, DIM), jnp.float32),
            pltpu.VMEM((BLK, HALF), jnp.float32),
            pltpu.VMEM((BLK, HALF), jnp.float32),
            pltpu.SemaphoreType.DMA,
            pltpu.SemaphoreType.DMA,
            pltpu.SemaphoreType.DMA,
            pltpu.SemaphoreType.DMA,
        ],
        compiler_params=pltpu.CompilerParams(needs_layout_passes=False),
    )(_permute_body)
    return f(x, perm)


def kernel(x, shuffled_indices, inverse_indices, reverse):
    perm = jnp.where(jnp.asarray(reverse), inverse_indices, shuffled_indices)
    y = _permute(x, perm)
    objective = jnp.zeros((), dtype=jnp.float32)
    return (y, objective)
